# Initial kernel scaffold; baseline (speedup 1.0000x reference)
import jax
import jax.numpy as jnp
from jax.experimental import pallas as pl


def kernel(x, edge_index, W1, a1_src, a1_dst, b1, W2, a2_src, a2_dst, b2):
    raise NotImplementedError("write your pallas kernel here")


# speedup vs baseline: 67.1360x; 67.1360x over previous
import functools

import jax
import jax.numpy as jnp
from jax import lax
from jax.experimental import pallas as pl
from jax.experimental.pallas import tpu as pltpu
from jax.experimental.pallas import tpu_sc as plsc

_CHUNK = 128
_IB = 2
_NSC = 2
_NTILE = 16
_NWORK = _NSC * _NTILE


def _edge_body(src2d, dst2d, f0t, f1t, asn, adn, zt, out,
               sidx, didx, f0b, f1b, asb, adb, es, ex, ey,
               accs, accx, accy, gsem, ssem,
               *, cpw, nblk, rpt):
    c = lax.axis_index("c")
    s = lax.axis_index("s")
    wid = s * _NSC + c

    zsl = pl.ds(s * rpt, rpt)
    pltpu.sync_copy(zt, accs.at[zsl])
    pltpu.sync_copy(zt, accx.at[zsl])
    pltpu.sync_copy(zt, accy.at[zsl])
    plsc.subcore_barrier()

    base_chunk = wid * cpw

    def block(b, carry):
        row0 = base_chunk + b * _IB
        pltpu.sync_copy(src2d.at[pl.ds(row0, _IB)], sidx)
        pltpu.sync_copy(dst2d.at[pl.ds(row0, _IB)], didx)
        descs = []
        for j in range(_IB):
            descs.append(pltpu.async_copy(f0t.at[sidx.at[j]], f0b.at[j], gsem))
            descs.append(pltpu.async_copy(f1t.at[sidx.at[j]], f1b.at[j], gsem))
            descs.append(pltpu.async_copy(asn.at[sidx.at[j]], asb.at[j], gsem))
            descs.append(pltpu.async_copy(adn.at[didx.at[j]], adb.at[j], gsem))
        for d in descs:
            d.wait()
        for j in range(_IB):
            for g in range(_CHUNK // 16):
                sl = pl.ds(g * 16, 16)
                av = asb[j, sl] + adb[j, sl]
                av = jnp.where(av >= 0.0, av, av * 0.2)
                e = jnp.exp(av)
                es[j, sl] = e
                ex[j, sl] = e * f0b[j, sl]
                ey[j, sl] = e * f1b[j, sl]
        sdescs = []
        for j in range(_IB):
            sdescs.append(
                pltpu.async_copy(es.at[j], accs.at[didx.at[j]], ssem, add=True))
            sdescs.append(
                pltpu.async_copy(ex.at[j], accx.at[didx.at[j]], ssem, add=True))
            sdescs.append(
                pltpu.async_copy(ey.at[j], accy.at[didx.at[j]], ssem, add=True))
        for d in sdescs:
            d.wait()
        return carry

    lax.fori_loop(0, nblk, block, 0)

    plsc.subcore_barrier()
    np_total = rpt * _NTILE
    obase = c * (3 * np_total) + s * rpt
    pltpu.sync_copy(accs.at[zsl], out.at[pl.ds(obase, rpt)])
    pltpu.sync_copy(accx.at[zsl], out.at[pl.ds(obase + np_total, rpt)])
    pltpu.sync_copy(accy.at[zsl], out.at[pl.ds(obase + 2 * np_total, rpt)])


def _make_edge_call(NP, cpw, rpt):
    mesh = plsc.VectorSubcoreMesh(core_axis_name="c", subcore_axis_name="s")
    body = functools.partial(_edge_body, cpw=cpw, nblk=cpw // _IB, rpt=rpt)
    ibuf = lambda: pltpu.VMEM((_IB, _CHUNK), jnp.int32)
    fbuf = lambda: pltpu.VMEM((_IB, _CHUNK), jnp.float32)
    return pl.kernel(
        body,
        out_type=jax.ShapeDtypeStruct((_NSC * 3 * NP,), jnp.float32),
        mesh=mesh,
        scratch_types=[
            ibuf(), ibuf(),
            fbuf(), fbuf(), fbuf(), fbuf(), fbuf(), fbuf(), fbuf(),
            pltpu.VMEM_SHARED((NP,), jnp.float32),
            pltpu.VMEM_SHARED((NP,), jnp.float32),
            pltpu.VMEM_SHARED((NP,), jnp.float32),
            pltpu.SemaphoreType.DMA,
            pltpu.SemaphoreType.DMA,
        ],
    )


def _pre_body(x0r, x1r, W1r, a1sr, a1dr, asr, adr):
    vs0 = jnp.sum(W1r[0, :] * a1sr[...])
    vs1 = jnp.sum(W1r[1, :] * a1sr[...])
    vd0 = jnp.sum(W1r[0, :] * a1dr[...])
    vd1 = jnp.sum(W1r[1, :] * a1dr[...])
    asr[...] = x0r[...] * vs0 + x1r[...] * vs1
    adr[...] = x0r[...] * vd0 + x1r[...] * vd1


def _mid_body(sar, sbr, gxar, gxbr, gyar, gybr, W1r, b1r, w2ar, w2br,
              a2sr, a2dr, h0r, h1r, as2r, ad2r):
    inv = 1.0 / (sar[...] + sbr[...] + 1e-16)
    g0 = (gxar[...] + gxbr[...]) * inv
    g1 = (gyar[...] + gybr[...]) * inv
    u = g0[:, None] * W1r[0, :][None, :] + g1[:, None] * W1r[1, :][None, :]
    u = jnp.maximum(u + b1r[...][None, :], 0.0)
    h0 = jnp.sum(u * w2ar[...][None, :], axis=1)
    h1 = jnp.sum(u * w2br[...][None, :], axis=1)
    h0r[...] = h0
    h1r[...] = h1
    as2r[...] = h0 * a2sr[0] + h1 * a2sr[1]
    ad2r[...] = h0 * a2dr[0] + h1 * a2dr[1]


def _post_body(sar, sbr, oxar, oxbr, oyar, oybr, b2r, y0r, y1r):
    inv = 1.0 / (sar[...] + sbr[...] + 1e-16)
    o0 = (oxar[...] + oxbr[...]) * inv + b2r[0]
    o1 = (oyar[...] + oybr[...]) * inv + b2r[1]
    m = jnp.maximum(o0, o1)
    lse = m + jnp.log(jnp.exp(o0 - m) + jnp.exp(o1 - m))
    y0r[...] = o0 - lse
    y1r[...] = o1 - lse


def kernel(x, edge_index, W1, a1_src, a1_dst, b1, W2, a2_src, a2_dst, b2):
    N = x.shape[0]
    E = edge_index.shape[1]
    src = edge_index[0].astype(jnp.int32)
    dst = edge_index[1].astype(jnp.int32)

    chunks = -(-E // _CHUNK)
    cpw = -(-chunks // _NWORK)
    cpw = -(-cpw // _IB) * _IB
    Ep = cpw * _NWORK * _CHUNK
    pad = Ep - E
    srcp = jnp.concatenate([src, jnp.zeros((pad,), jnp.int32)])
    dstp = jnp.concatenate([dst, jnp.full((pad,), N, jnp.int32)])
    src2d = srcp.reshape(-1, _CHUNK)
    dst2d = dstp.reshape(-1, _CHUNK)

    rpt = -(-(N + 1) // _NTILE)
    rpt = -(-rpt // 128) * 128
    NP = rpt * _NTILE
    zt = jnp.zeros((rpt,), jnp.float32)

    edge_call = _make_edge_call(NP, cpw, rpt)

    x0 = x[:, 0]
    x1 = x[:, 1]
    as1, ad1 = pl.pallas_call(
        _pre_body,
        out_shape=[jax.ShapeDtypeStruct((N,), jnp.float32)] * 2,
    )(x0, x1, W1, a1_src, a1_dst)
    ad1p = jnp.concatenate([ad1, jnp.zeros((16,), jnp.float32)])

    acc1 = edge_call(src2d, dst2d, x0, x1, as1, ad1p, zt).reshape(_NSC, 3, NP)

    Bn = 8192
    nblocks = -(-N // Bn)
    nspec = pl.BlockSpec((Bn,), lambda i: (i,))
    vspec64 = pl.BlockSpec((64,), lambda i: (0,))
    sspec = pl.BlockSpec(memory_space=pltpu.SMEM)
    h0, h1, as2, ad2 = pl.pallas_call(
        _mid_body,
        grid=(nblocks,),
        in_specs=[nspec] * 6 + [pl.BlockSpec((2, 64), lambda i: (0, 0)),
                                vspec64, vspec64, vspec64, sspec, sspec],
        out_specs=[nspec] * 4,
        out_shape=[jax.ShapeDtypeStruct((N,), jnp.float32)] * 4,
    )(acc1[0, 0, :N], acc1[1, 0, :N], acc1[0, 1, :N], acc1[1, 1, :N],
      acc1[0, 2, :N], acc1[1, 2, :N], W1, b1, W2[:, 0], W2[:, 1],
      a2_src, a2_dst)

    ad2p = jnp.concatenate([ad2, jnp.zeros((16,), jnp.float32)])

    acc2 = edge_call(src2d, dst2d, h0, h1, as2, ad2p, zt).reshape(_NSC, 3, NP)

    y0, y1 = pl.pallas_call(
        _post_body,
        grid=(nblocks,),
        in_specs=[nspec] * 6 + [sspec],
        out_specs=[nspec] * 2,
        out_shape=[jax.ShapeDtypeStruct((N,), jnp.float32)] * 2,
    )(acc2[0, 0, :N], acc2[1, 0, :N], acc2[0, 1, :N], acc2[1, 1, :N],
      acc2[0, 2, :N], acc2[1, 2, :N], b2)

    return jnp.stack([y0, y1], axis=1)
